# baseline (speedup 1.0000x reference)
import functools

import jax
import jax.numpy as jnp
from jax import lax
from jax.experimental import pallas as pl
from jax.experimental.pallas import tpu as pltpu
from jax.experimental.pallas import tpu_sc as plsc

N_NODES = 10000
N_EDGES = 160000
IN_F = 256
NH = 8
HD = 64
F = NH * HD
NEG = 0.2

NC, NS = 2, 16
NT = NC * NS
CH = 128
CPT = 40
EP = NT * CPT * CH
NPR = 632
NP_ = NS * NPR
NPB = 4
PW = 128
BN = 1000


def _wl_body(wt_ref, ap_ref, wlr_ref):
    wlr_ref[...] = jnp.dot(wt_ref[...], ap_ref[...],
                           preferred_element_type=jnp.float32)


def _tc_wl(Wt, ALPARP):
    return pl.pallas_call(
        _wl_body,
        out_shape=jax.ShapeDtypeStruct((IN_F, 32), jnp.float32),
    )(Wt, ALPARP)


def _eler_body(feat_ref, wlr_ref, elp_ref, erp_ref):
    f = feat_ref[...]
    elp_ref[...] = jnp.dot(f, wlr_ref[:, :16],
                           preferred_element_type=jnp.float32)
    erp_ref[...] = jnp.dot(f, wlr_ref[:, 16:],
                           preferred_element_type=jnp.float32)


def _tc_eler(feat, wlr):
    sds = jax.ShapeDtypeStruct
    return pl.pallas_call(
        _eler_body,
        grid=(N_NODES // BN,),
        in_specs=[
            pl.BlockSpec((BN, IN_F), lambda i: (i, 0)),
            pl.BlockSpec((IN_F, 32), lambda i: (0, 0)),
        ],
        out_specs=[
            pl.BlockSpec((BN, 16), lambda i: (i, 0)),
            pl.BlockSpec((BN, 16), lambda i: (i, 0)),
        ],
        out_shape=[
            sds((N_NODES, 16), jnp.float32),
            sds((N_NODES, 16), jnp.float32),
        ],
    )(feat, wlr)


def _proj_body(feat_ref, w4_ref, proj_ref):
    proj_ref[0] = jnp.dot(feat_ref[...], w4_ref[0],
                          preferred_element_type=jnp.float32)


def _tc_project(feat, Wt4):
    return pl.pallas_call(
        _proj_body,
        grid=(N_NODES // BN, NPB),
        in_specs=[
            pl.BlockSpec((BN, IN_F), lambda i, p: (i, 0)),
            pl.BlockSpec((1, IN_F, PW), lambda i, p: (p, 0, 0)),
        ],
        out_specs=pl.BlockSpec((1, BN, PW), lambda i, p: (p, i, 0)),
        out_shape=jax.ShapeDtypeStruct((NPB, N_NODES, PW), jnp.float32),
    )(feat, Wt4)


def _ka_body(elp, erp, srcf, dst2d, w_out, dparts,
             src_v, dst_v, elba, elbb, erba, erbb, zb, denom_sh,
             sea, seb, sra, srb, swa, swb, ssa, ssb):
    c = lax.axis_index("c")
    s = lax.axis_index("s")
    t = c * NS + s

    @pl.loop(0, NPR)
    def _(j):
        zb[j] = jnp.zeros((16,), jnp.float32)

    pltpu.sync_copy(zb, denom_sh.at[pl.ds(s * NPR, NPR)])
    plsc.subcore_barrier()

    pltpu.sync_copy(srcf.at[pl.ds(t * (CPT * CH), CPT * CH)], src_v)
    pltpu.sync_copy(dst2d.at[pl.ds(t * CPT, CPT)], dst_v)

    def fire(k, eb, rb, se, sr):
        pltpu.async_copy(elp.at[src_v.at[pl.ds(k * CH, CH)]], eb, se)
        pltpu.async_copy(erp.at[dst_v.at[k]], rb, sr)

    def wait_in(k, eb, rb, se, sr):
        pltpu.make_async_copy(elp.at[src_v.at[pl.ds(k * CH, CH)]],
                              eb, se).wait()
        pltpu.make_async_copy(erp.at[dst_v.at[k]], rb, sr).wait()

    def compute(eb, rb):
        @plsc.parallel_loop(0, CH, unroll=8)
        def _(j):
            x = eb[j] + rb[j]
            x = jnp.maximum(x, NEG * x)
            eb[j] = jnp.exp(x)

    fire(0, elba, erba, sea, sra)
    fire(1, elbb, erbb, seb, srb)

    @pl.loop(0, CPT // 2)
    def _(i):
        kA = 2 * i
        kB = kA + 1
        wait_in(kA, elba, erba, sea, sra)
        compute(elba, erba)
        pltpu.async_copy(elba, w_out.at[pl.ds((t * CPT + kA) * CH, CH)], swa)
        pltpu.async_copy(elba, denom_sh.at[dst_v.at[kA]], ssa, add=True)
        wait_in(kB, elbb, erbb, seb, srb)
        compute(elbb, erbb)
        pltpu.async_copy(elbb, w_out.at[pl.ds((t * CPT + kB) * CH, CH)], swb)
        pltpu.async_copy(elbb, denom_sh.at[dst_v.at[kB]], ssb, add=True)

        pltpu.make_async_copy(
            elba, w_out.at[pl.ds((t * CPT + kA) * CH, CH)], swa).wait()
        pltpu.make_async_copy(elba, denom_sh.at[dst_v.at[kA]], ssa).wait()

        @pl.when(kA + 2 < CPT)
        def _():
            fire(kA + 2, elba, erba, sea, sra)

        pltpu.make_async_copy(
            elbb, w_out.at[pl.ds((t * CPT + kB) * CH, CH)], swb).wait()
        pltpu.make_async_copy(elbb, denom_sh.at[dst_v.at[kB]], ssb).wait()

        @pl.when(kB + 2 < CPT)
        def _():
            fire(kB + 2, elbb, erbb, seb, srb)

    plsc.subcore_barrier()
    pltpu.sync_copy(denom_sh.at[pl.ds(s * NPR, NPR)],
                    dparts.at[c, pl.ds(s * NPR, NPR)])


def _kb_body(proj2, srcf, dst2d, w_hbm, dparts, out4,
             src_v, dst_v, gbufa, gbufb, wbufa, wbufb, rst_sh,
             sga, sgb, swa, swb, ssa, ssb):
    c = lax.axis_index("c")
    s = lax.axis_index("s")
    kpt = EP // CH // NS
    kph = kpt // 2

    for pp in range(2):
        p = c * 2 + pp

        @pl.loop(0, CH)
        def _(j):
            for kk in range(8):
                gbufa[j, pl.ds(16 * kk, 16)] = jnp.zeros((16,), jnp.float32)

        off = 0
        for sz in (128, 128, 128, 128, 120):
            pltpu.sync_copy(gbufa.at[pl.ds(0, sz)],
                            rst_sh.at[pl.ds(s * NPR + off, sz)])
            off += sz
        plsc.subcore_barrier()

        pN = p * N_NODES
        lane0 = jnp.full((16,), 2 * p, jnp.int32)
        lane1 = lane0 + 1

        def scale(gb, wb):
            @plsc.parallel_loop(0, CH, unroll=8)
            def _(j):
                wrow = wb[j]
                b0 = wrow.at[lane0].get(mode="promise_in_bounds")
                b1 = wrow.at[lane1].get(mode="promise_in_bounds")
                for kk in range(4):
                    gb[j, pl.ds(16 * kk, 16)] *= b0
                for kk in range(4, 8):
                    gb[j, pl.ds(16 * kk, 16)] *= b1

        for h in range(2):
            g0 = s * kpt + h * kph
            pltpu.sync_copy(srcf.at[pl.ds(g0 * CH, kph * CH)], src_v)
            pltpu.sync_copy(dst2d.at[pl.ds(g0, kph)], dst_v)

            @pl.loop(0, kph * CH // 16)
            def _(j):
                src_v[pl.ds(j * 16, 16)] += pN

            def fire(k, gb, wb, sg, sw):
                pltpu.async_copy(proj2.at[src_v.at[pl.ds(k * CH, CH)]],
                                 gb, sg)
                pltpu.async_copy(w_hbm.at[pl.ds((g0 + k) * CH, CH)], wb, sw)

            def wait_in(k, gb, wb, sg, sw):
                pltpu.make_async_copy(
                    proj2.at[src_v.at[pl.ds(k * CH, CH)]], gb, sg).wait()
                pltpu.make_async_copy(
                    w_hbm.at[pl.ds((g0 + k) * CH, CH)], wb, sw).wait()

            fire(0, gbufa, wbufa, sga, swa)
            fire(1, gbufb, wbufb, sgb, swb)

            @pl.loop(0, kph // 2)
            def _(i):
                kA = 2 * i
                kB = kA + 1
                wait_in(kA, gbufa, wbufa, sga, swa)
                scale(gbufa, wbufa)
                pltpu.async_copy(gbufa, rst_sh.at[dst_v.at[kA]], ssa,
                                 add=True)
                wait_in(kB, gbufb, wbufb, sgb, swb)
                scale(gbufb, wbufb)
                pltpu.async_copy(gbufb, rst_sh.at[dst_v.at[kB]], ssb,
                                 add=True)
                pltpu.make_async_copy(gbufa, rst_sh.at[dst_v.at[kA]],
                                      ssa).wait()

                @pl.when(kA + 2 < kph)
                def _():
                    fire(kA + 2, gbufa, wbufa, sga, swa)

                pltpu.make_async_copy(gbufb, rst_sh.at[dst_v.at[kB]],
                                      ssb).wait()

                @pl.when(kB + 2 < kph)
                def _():
                    fire(kB + 2, gbufb, wbufb, sgb, swb)

        plsc.subcore_barrier()

        base = 0
        for sz in (80, 80, 80, 80, 80, 80, 80, 72):
            r0 = s * NPR + base
            pltpu.sync_copy(rst_sh.at[pl.ds(r0, sz)], gbufa.at[pl.ds(0, sz)])
            pltpu.sync_copy(dparts.at[0, pl.ds(r0, sz)], wbufa.at[pl.ds(0, sz)])
            pltpu.sync_copy(dparts.at[1, pl.ds(r0, sz)], wbufb.at[pl.ds(0, sz)])

            @plsc.parallel_loop(0, sz, unroll=4)
            def _(j):
                drow = wbufa[j] + wbufb[j]
                d0 = drow.at[lane0].get(mode="promise_in_bounds")
                d1 = drow.at[lane1].get(mode="promise_in_bounds")
                d0c = jnp.maximum(d0, 1e-30)
                d1c = jnp.maximum(d1, 1e-30)
                for kk in range(8):
                    d = d0c if kk < 4 else d1c
                    v = gbufa[j, pl.ds(16 * kk, 16)]
                    gbufa[j, pl.ds(16 * kk, 16)] = v / d

            pltpu.sync_copy(gbufa.at[pl.ds(0, sz)], out4.at[p, pl.ds(r0, sz)])
            base += sz

        plsc.subcore_barrier()


def _sc_mesh():
    return plsc.VectorSubcoreMesh(core_axis_name="c", subcore_axis_name="s",
                                  num_cores=NC, num_subcores=NS)


_SC_PARAMS = pltpu.CompilerParams(use_tc_tiling_on_sc=False)


def _run_ka(elp_p, erp_p, srcf, dst2d):
    sds = jax.ShapeDtypeStruct
    f = pl.kernel(
        _ka_body,
        out_type=(sds((EP, 16), jnp.float32), sds((NC, NP_, 16), jnp.float32)),
        mesh=_sc_mesh(),
        scratch_types=[
            pltpu.VMEM((CPT * CH,), jnp.int32),
            pltpu.VMEM((CPT, CH), jnp.int32),
            pltpu.VMEM((CH, 16), jnp.float32),
            pltpu.VMEM((CH, 16), jnp.float32),
            pltpu.VMEM((CH, 16), jnp.float32),
            pltpu.VMEM((CH, 16), jnp.float32),
            pltpu.VMEM((NPR, 16), jnp.float32),
            pltpu.VMEM_SHARED((NP_, 16), jnp.float32),
        ] + [pltpu.SemaphoreType.DMA] * 8,
        compiler_params=_SC_PARAMS,
    )
    return f(elp_p, erp_p, srcf, dst2d)


def _run_kb(proj2, srcf, dst2d, w_e, dparts):
    sds = jax.ShapeDtypeStruct
    kpt = EP // CH // NS
    f = pl.kernel(
        _kb_body,
        out_type=sds((NPB, NP_, PW), jnp.float32),
        mesh=_sc_mesh(),
        scratch_types=[
            pltpu.VMEM((kpt * CH // 2,), jnp.int32),
            pltpu.VMEM((kpt // 2, CH), jnp.int32),
            pltpu.VMEM((CH, PW), jnp.float32),
            pltpu.VMEM((CH, PW), jnp.float32),
            pltpu.VMEM((CH, 16), jnp.float32),
            pltpu.VMEM((CH, 16), jnp.float32),
            pltpu.VMEM_SHARED((NP_, PW), jnp.float32),
            pltpu.SemaphoreType.DMA,
            pltpu.SemaphoreType.DMA,
            pltpu.SemaphoreType.DMA,
            pltpu.SemaphoreType.DMA,
            pltpu.SemaphoreType.DMA,
            pltpu.SemaphoreType.DMA,
        ],
        compiler_params=_SC_PARAMS,
    )
    return f(proj2, srcf, dst2d, w_e, dparts)


def kernel(feat, edge_index, W_fc, attn_l, attn_r):
    Wt = W_fc.T
    Wt4 = Wt.reshape(IN_F, NPB, PW).transpose(1, 0, 2)
    eye = jnp.eye(NH, dtype=jnp.float32)
    al = attn_l.reshape(NH, HD)
    ar = attn_r.reshape(NH, HD)
    ALP = jnp.pad((al[:, :, None] * eye[:, None, :]).reshape(F, NH),
                  ((0, 0), (0, 8)))
    ARP = jnp.pad((ar[:, :, None] * eye[:, None, :]).reshape(F, NH),
                  ((0, 0), (0, 8)))
    ALPARP = jnp.concatenate([ALP, ARP], axis=1)

    wlr = _tc_wl(Wt, ALPARP)
    elp, erp = _tc_eler(feat, wlr)
    elp_p = jnp.pad(elp, ((0, NP_ - N_NODES), (0, 0)))
    erp_p = jnp.pad(erp, ((0, NP_ - N_NODES), (0, 0)))
    proj4 = _tc_project(feat, Wt4)
    proj2 = proj4.reshape(NPB * N_NODES, PW)

    srcf = jnp.pad(edge_index[0], (0, EP - N_EDGES))
    dstf = jnp.pad(edge_index[1], (0, EP - N_EDGES), constant_values=N_NODES)
    dst2d = dstf.reshape(EP // CH, CH)

    w_e, dparts = _run_ka(elp_p, erp_p, srcf, dst2d)
    out4 = _run_kb(proj2, srcf, dst2d, w_e, dparts)
    return out4[:, :N_NODES].transpose(1, 0, 2).reshape(N_NODES, NH, HD)


# speedup vs baseline: 1.2473x; 1.2473x over previous
import functools

import jax
import jax.numpy as jnp
from jax import lax
from jax.experimental import pallas as pl
from jax.experimental.pallas import tpu as pltpu
from jax.experimental.pallas import tpu_sc as plsc

N_NODES = 10000
N_EDGES = 160000
IN_F = 256
NH = 8
HD = 64
F = NH * HD
NEG = 0.2

NC, NS = 2, 16
NT = NC * NS
CH = 128
CPT = 40
EP = NT * CPT * CH
NPR = 632
NP_ = NS * NPR
NPB = 4
PW = 128
BN = 1000


def _wl_body(wt_ref, ap_ref, wlr_ref):
    wlr_ref[...] = jnp.dot(wt_ref[...], ap_ref[...],
                           preferred_element_type=jnp.float32)


def _tc_wl(Wt, ALPARP):
    return pl.pallas_call(
        _wl_body,
        out_shape=jax.ShapeDtypeStruct((IN_F, 32), jnp.float32),
    )(Wt, ALPARP)


def _eler_body(feat_ref, wlr_ref, elp_ref, erp_ref):
    f = feat_ref[...]
    elp_ref[...] = jnp.dot(f, wlr_ref[:, :16],
                           preferred_element_type=jnp.float32)
    erp_ref[...] = jnp.dot(f, wlr_ref[:, 16:],
                           preferred_element_type=jnp.float32)


def _tc_eler(feat, wlr):
    sds = jax.ShapeDtypeStruct
    return pl.pallas_call(
        _eler_body,
        grid=(N_NODES // BN,),
        in_specs=[
            pl.BlockSpec((BN, IN_F), lambda i: (i, 0)),
            pl.BlockSpec((IN_F, 32), lambda i: (0, 0)),
        ],
        out_specs=[
            pl.BlockSpec((BN, 16), lambda i: (i, 0)),
            pl.BlockSpec((BN, 16), lambda i: (i, 0)),
        ],
        out_shape=[
            sds((N_NODES, 16), jnp.float32),
            sds((N_NODES, 16), jnp.float32),
        ],
    )(feat, wlr)


def _proj_body(feat_ref, w8_ref, proj_ref):
    proj_ref[0] = jnp.dot(feat_ref[...], w8_ref[0],
                          preferred_element_type=jnp.float32)


def _tc_project(feat, W8):
    return pl.pallas_call(
        _proj_body,
        grid=(N_NODES // BN, NH),
        in_specs=[
            pl.BlockSpec((BN, IN_F), lambda i, h: (i, 0)),
            pl.BlockSpec((1, IN_F, HD), lambda i, h: (h, 0, 0)),
        ],
        out_specs=pl.BlockSpec((1, BN, HD), lambda i, h: (h, i, 0)),
        out_shape=jax.ShapeDtypeStruct((NH, N_NODES, HD), jnp.float32),
    )(feat, W8)


def _ka_body(elp, erp, srcf, dst2d, w_out, dparts,
             src_v, dst_v, elba, elbb, erba, erbb, zb, denom_sh,
             sea, seb, sra, srb, swa, swb, ssa, ssb):
    c = lax.axis_index("c")
    s = lax.axis_index("s")
    t = c * NS + s

    @pl.loop(0, NPR)
    def _(j):
        zb[j] = jnp.zeros((16,), jnp.float32)

    pltpu.sync_copy(zb, denom_sh.at[pl.ds(s * NPR, NPR)])
    plsc.subcore_barrier()

    pltpu.sync_copy(srcf.at[pl.ds(t * (CPT * CH), CPT * CH)], src_v)
    pltpu.sync_copy(dst2d.at[pl.ds(t * CPT, CPT)], dst_v)

    def fire(k, eb, rb, se, sr):
        pltpu.async_copy(elp.at[src_v.at[pl.ds(k * CH, CH)]], eb, se)
        pltpu.async_copy(erp.at[dst_v.at[k]], rb, sr)

    def wait_in(k, eb, rb, se, sr):
        pltpu.make_async_copy(elp.at[src_v.at[pl.ds(k * CH, CH)]],
                              eb, se).wait()
        pltpu.make_async_copy(erp.at[dst_v.at[k]], rb, sr).wait()

    def compute(eb, rb):
        @plsc.parallel_loop(0, CH, unroll=8)
        def _(j):
            x = eb[j] + rb[j]
            x = jnp.maximum(x, NEG * x)
            eb[j] = jnp.exp(x)

    fire(0, elba, erba, sea, sra)
    fire(1, elbb, erbb, seb, srb)

    @pl.loop(0, CPT // 2)
    def _(i):
        kA = 2 * i
        kB = kA + 1
        wait_in(kA, elba, erba, sea, sra)
        compute(elba, erba)
        pltpu.async_copy(elba, w_out.at[pl.ds((t * CPT + kA) * CH, CH)], swa)
        pltpu.async_copy(elba, denom_sh.at[dst_v.at[kA]], ssa, add=True)
        wait_in(kB, elbb, erbb, seb, srb)
        compute(elbb, erbb)
        pltpu.async_copy(elbb, w_out.at[pl.ds((t * CPT + kB) * CH, CH)], swb)
        pltpu.async_copy(elbb, denom_sh.at[dst_v.at[kB]], ssb, add=True)

        pltpu.make_async_copy(
            elba, w_out.at[pl.ds((t * CPT + kA) * CH, CH)], swa).wait()
        pltpu.make_async_copy(elba, denom_sh.at[dst_v.at[kA]], ssa).wait()

        @pl.when(kA + 2 < CPT)
        def _():
            fire(kA + 2, elba, erba, sea, sra)

        pltpu.make_async_copy(
            elbb, w_out.at[pl.ds((t * CPT + kB) * CH, CH)], swb).wait()
        pltpu.make_async_copy(elbb, denom_sh.at[dst_v.at[kB]], ssb).wait()

        @pl.when(kB + 2 < CPT)
        def _():
            fire(kB + 2, elbb, erbb, seb, srb)

    plsc.subcore_barrier()
    pltpu.sync_copy(denom_sh.at[pl.ds(s * NPR, NPR)],
                    dparts.at[c, pl.ds(s * NPR, NPR)])


def _kb_body(proj8, srcf, dst2d, w_hbm, dparts, out8,
             src_v, dst_v, gbufa, gbufb, wbufa, wbufb, table_sh, rst_sh,
             sga, sgb, swa, swb, ssa, ssb):
    c = lax.axis_index("c")
    s = lax.axis_index("s")
    kpt = EP // CH // NS

    pltpu.sync_copy(srcf.at[pl.ds(s * (kpt * CH), kpt * CH)], src_v)
    pltpu.sync_copy(dst2d.at[pl.ds(s * kpt, kpt)], dst_v)

    def fire(k, gb, wb, sg, sw):
        pltpu.async_copy(table_sh.at[src_v.at[pl.ds(k * CH, CH)]], gb, sg)
        pltpu.async_copy(w_hbm.at[pl.ds((s * kpt + k) * CH, CH)], wb, sw)

    def wait_in(k, gb, wb, sg, sw):
        pltpu.make_async_copy(table_sh.at[src_v.at[pl.ds(k * CH, CH)]],
                              gb, sg).wait()
        pltpu.make_async_copy(w_hbm.at[pl.ds((s * kpt + k) * CH, CH)],
                              wb, sw).wait()

    for hh in range(NH // NC):
        h = c * (NH // NC) + hh
        lane_h = jnp.full((16,), h, jnp.int32)

        pltpu.sync_copy(proj8.at[h, pl.ds(s * 625, 625)],
                        table_sh.at[pl.ds(s * 625, 625)])

        @pl.loop(0, CH)
        def _(j):
            for kk in range(4):
                gbufa[j, pl.ds(16 * kk, 16)] = jnp.zeros((16,), jnp.float32)

        off = 0
        for sz in (128, 128, 128, 128, 120):
            pltpu.sync_copy(gbufa.at[pl.ds(0, sz)],
                            rst_sh.at[pl.ds(s * NPR + off, sz)])
            off += sz
        plsc.subcore_barrier()

        def scale(gb, wb):
            @plsc.parallel_loop(0, CH, unroll=8)
            def _(j):
                wrow = wb[j]
                b = wrow.at[lane_h].get(mode="promise_in_bounds")
                for kk in range(4):
                    gb[j, pl.ds(16 * kk, 16)] *= b

        fire(0, gbufa, wbufa, sga, swa)
        fire(1, gbufb, wbufb, sgb, swb)

        @pl.loop(0, kpt // 2)
        def _(i):
            kA = 2 * i
            kB = kA + 1
            wait_in(kA, gbufa, wbufa, sga, swa)
            scale(gbufa, wbufa)
            pltpu.async_copy(gbufa, rst_sh.at[dst_v.at[kA]], ssa, add=True)
            wait_in(kB, gbufb, wbufb, sgb, swb)
            scale(gbufb, wbufb)
            pltpu.async_copy(gbufb, rst_sh.at[dst_v.at[kB]], ssb, add=True)
            pltpu.make_async_copy(gbufa, rst_sh.at[dst_v.at[kA]], ssa).wait()

            @pl.when(kA + 2 < kpt)
            def _():
                fire(kA + 2, gbufa, wbufa, sga, swa)

            pltpu.make_async_copy(gbufb, rst_sh.at[dst_v.at[kB]], ssb).wait()

            @pl.when(kB + 2 < kpt)
            def _():
                fire(kB + 2, gbufb, wbufb, sgb, swb)

        plsc.subcore_barrier()

        base = 0
        for sz in (80, 80, 80, 80, 80, 80, 80, 72):
            r0 = s * NPR + base
            pltpu.sync_copy(rst_sh.at[pl.ds(r0, sz)], gbufa.at[pl.ds(0, sz)])
            pltpu.sync_copy(dparts.at[0, pl.ds(r0, sz)], wbufa.at[pl.ds(0, sz)])
            pltpu.sync_copy(dparts.at[1, pl.ds(r0, sz)], wbufb.at[pl.ds(0, sz)])

            @plsc.parallel_loop(0, sz, unroll=4)
            def _(j):
                drow = wbufa[j] + wbufb[j]
                d = jnp.maximum(
                    drow.at[lane_h].get(mode="promise_in_bounds"), 1e-30)
                for kk in range(4):
                    v = gbufa[j, pl.ds(16 * kk, 16)]
                    gbufa[j, pl.ds(16 * kk, 16)] = v / d

            pltpu.sync_copy(gbufa.at[pl.ds(0, sz)], out8.at[h, pl.ds(r0, sz)])
            base += sz

        plsc.subcore_barrier()


def _sc_mesh():
    return plsc.VectorSubcoreMesh(core_axis_name="c", subcore_axis_name="s",
                                  num_cores=NC, num_subcores=NS)


_SC_PARAMS = pltpu.CompilerParams(use_tc_tiling_on_sc=False)


def _run_ka(elp_p, erp_p, srcf, dst2d):
    sds = jax.ShapeDtypeStruct
    f = pl.kernel(
        _ka_body,
        out_type=(sds((EP, 16), jnp.float32), sds((NC, NP_, 16), jnp.float32)),
        mesh=_sc_mesh(),
        scratch_types=[
            pltpu.VMEM((CPT * CH,), jnp.int32),
            pltpu.VMEM((CPT, CH), jnp.int32),
            pltpu.VMEM((CH, 16), jnp.float32),
            pltpu.VMEM((CH, 16), jnp.float32),
            pltpu.VMEM((CH, 16), jnp.float32),
            pltpu.VMEM((CH, 16), jnp.float32),
            pltpu.VMEM((NPR, 16), jnp.float32),
            pltpu.VMEM_SHARED((NP_, 16), jnp.float32),
        ] + [pltpu.SemaphoreType.DMA] * 8,
        compiler_params=_SC_PARAMS,
    )
    return f(elp_p, erp_p, srcf, dst2d)


def _run_kb(proj8, srcf, dst2d, w_e, dparts):
    sds = jax.ShapeDtypeStruct
    kpt = EP // CH // NS
    f = pl.kernel(
        _kb_body,
        out_type=sds((NH, NP_, HD), jnp.float32),
        mesh=_sc_mesh(),
        scratch_types=[
            pltpu.VMEM((kpt * CH,), jnp.int32),
            pltpu.VMEM((kpt, CH), jnp.int32),
            pltpu.VMEM((CH, HD), jnp.float32),
            pltpu.VMEM((CH, HD), jnp.float32),
            pltpu.VMEM((CH, 16), jnp.float32),
            pltpu.VMEM((CH, 16), jnp.float32),
            pltpu.VMEM_SHARED((N_NODES, HD), jnp.float32),
            pltpu.VMEM_SHARED((NP_, HD), jnp.float32),
        ] + [pltpu.SemaphoreType.DMA] * 6,
        compiler_params=_SC_PARAMS,
    )
    return f(proj8, srcf, dst2d, w_e, dparts)


def kernel(feat, edge_index, W_fc, attn_l, attn_r):
    Wt = W_fc.T
    W8 = Wt.reshape(IN_F, NH, HD).transpose(1, 0, 2)
    eye = jnp.eye(NH, dtype=jnp.float32)
    al = attn_l.reshape(NH, HD)
    ar = attn_r.reshape(NH, HD)
    ALP = jnp.pad((al[:, :, None] * eye[:, None, :]).reshape(F, NH),
                  ((0, 0), (0, 8)))
    ARP = jnp.pad((ar[:, :, None] * eye[:, None, :]).reshape(F, NH),
                  ((0, 0), (0, 8)))
    ALPARP = jnp.concatenate([ALP, ARP], axis=1)

    wlr = _tc_wl(Wt, ALPARP)
    elp, erp = _tc_eler(feat, wlr)
    elp_p = jnp.pad(elp, ((0, NP_ - N_NODES), (0, 0)))
    erp_p = jnp.pad(erp, ((0, NP_ - N_NODES), (0, 0)))
    proj8 = _tc_project(feat, W8)

    srcf = jnp.pad(edge_index[0], (0, EP - N_EDGES))
    dstf = jnp.pad(edge_index[1], (0, EP - N_EDGES), constant_values=N_NODES)
    dst2d = dstf.reshape(EP // CH, CH)

    w_e, dparts = _run_ka(elp_p, erp_p, srcf, dst2d)
    out8 = _run_kb(proj8, srcf, dst2d, w_e, dparts)
    return out8[:, :N_NODES].transpose(1, 0, 2)


# speedup vs baseline: 1.2506x; 1.0026x over previous
import functools

import jax
import jax.numpy as jnp
from jax import lax
from jax.experimental import pallas as pl
from jax.experimental.pallas import tpu as pltpu
from jax.experimental.pallas import tpu_sc as plsc

N_NODES = 10000
N_EDGES = 160000
IN_F = 256
NH = 8
HD = 64
F = NH * HD
NEG = 0.2

NC, NS = 2, 16
NT = NC * NS
CH = 128
CPT = 40
EP = NT * CPT * CH
NPR = 632
NP_ = NS * NPR
NPB = 4
PW = 128
BN = 1000


def _wl_body(wt_ref, ap_ref, wlr_ref):
    wlr_ref[...] = jnp.dot(wt_ref[...], ap_ref[...],
                           preferred_element_type=jnp.float32)


def _tc_wl(Wt, ALPARP):
    return pl.pallas_call(
        _wl_body,
        out_shape=jax.ShapeDtypeStruct((IN_F, 32), jnp.float32),
    )(Wt, ALPARP)


def _eler_body(feat_ref, wlr_ref, elp_ref, erp_ref):
    f = feat_ref[...]
    elp_ref[...] = jnp.dot(f, wlr_ref[:, :16],
                           preferred_element_type=jnp.float32)
    erp_ref[...] = jnp.dot(f, wlr_ref[:, 16:],
                           preferred_element_type=jnp.float32)


def _tc_eler(feat, wlr):
    sds = jax.ShapeDtypeStruct
    return pl.pallas_call(
        _eler_body,
        grid=(N_NODES // BN,),
        in_specs=[
            pl.BlockSpec((BN, IN_F), lambda i: (i, 0)),
            pl.BlockSpec((IN_F, 32), lambda i: (0, 0)),
        ],
        out_specs=[
            pl.BlockSpec((BN, 16), lambda i: (i, 0)),
            pl.BlockSpec((BN, 16), lambda i: (i, 0)),
        ],
        out_shape=[
            sds((NP_, 16), jnp.float32),
            sds((NP_, 16), jnp.float32),
        ],
    )(feat, wlr)


def _proj_body(feat_ref, w8_ref, proj_ref):
    proj_ref[0] = jnp.dot(feat_ref[...], w8_ref[0],
                          preferred_element_type=jnp.float32)


def _tc_project(feat, W8):
    return pl.pallas_call(
        _proj_body,
        grid=(N_NODES // BN, NH),
        in_specs=[
            pl.BlockSpec((BN, IN_F), lambda i, h: (i, 0)),
            pl.BlockSpec((1, IN_F, HD), lambda i, h: (h, 0, 0)),
        ],
        out_specs=pl.BlockSpec((1, BN, HD), lambda i, h: (h, i, 0)),
        out_shape=jax.ShapeDtypeStruct((NH, N_NODES, HD), jnp.float32),
    )(feat, W8)


def _ka_body(elp, erp, srcf, dst2d, w_out, dparts,
             src_v, dst_v, elba, elbb, erba, erbb, zb, denom_sh,
             sea, seb, sra, srb, swa, swb, ssa, ssb):
    c = lax.axis_index("c")
    s = lax.axis_index("s")
    t = c * NS + s

    @pl.loop(0, NPR)
    def _(j):
        zb[j] = jnp.zeros((16,), jnp.float32)

    pltpu.sync_copy(zb, denom_sh.at[pl.ds(s * NPR, NPR)])
    plsc.subcore_barrier()

    pltpu.sync_copy(srcf.at[pl.ds(t * (CPT * CH), CPT * CH)], src_v)
    pltpu.sync_copy(dst2d.at[pl.ds(t * CPT, CPT)], dst_v)

    def fire(k, eb, rb, se, sr):
        pltpu.async_copy(elp.at[src_v.at[pl.ds(k * CH, CH)]], eb, se)
        pltpu.async_copy(erp.at[dst_v.at[k]], rb, sr)

    def wait_in(k, eb, rb, se, sr):
        pltpu.make_async_copy(elp.at[src_v.at[pl.ds(k * CH, CH)]],
                              eb, se).wait()
        pltpu.make_async_copy(erp.at[dst_v.at[k]], rb, sr).wait()

    def compute(eb, rb):
        @plsc.parallel_loop(0, CH, unroll=8)
        def _(j):
            x = eb[j] + rb[j]
            x = jnp.maximum(x, NEG * x)
            eb[j] = jnp.exp(x)

    fire(0, elba, erba, sea, sra)
    fire(1, elbb, erbb, seb, srb)

    @pl.loop(0, CPT // 2)
    def _(i):
        kA = 2 * i
        kB = kA + 1
        wait_in(kA, elba, erba, sea, sra)
        compute(elba, erba)
        pltpu.async_copy(elba, w_out.at[pl.ds((t * CPT + kA) * CH, CH)], swa)
        pltpu.async_copy(elba, denom_sh.at[dst_v.at[kA]], ssa, add=True)
        wait_in(kB, elbb, erbb, seb, srb)
        compute(elbb, erbb)
        pltpu.async_copy(elbb, w_out.at[pl.ds((t * CPT + kB) * CH, CH)], swb)
        pltpu.async_copy(elbb, denom_sh.at[dst_v.at[kB]], ssb, add=True)

        pltpu.make_async_copy(
            elba, w_out.at[pl.ds((t * CPT + kA) * CH, CH)], swa).wait()
        pltpu.make_async_copy(elba, denom_sh.at[dst_v.at[kA]], ssa).wait()

        @pl.when(kA + 2 < CPT)
        def _():
            fire(kA + 2, elba, erba, sea, sra)

        pltpu.make_async_copy(
            elbb, w_out.at[pl.ds((t * CPT + kB) * CH, CH)], swb).wait()
        pltpu.make_async_copy(elbb, denom_sh.at[dst_v.at[kB]], ssb).wait()

        @pl.when(kB + 2 < CPT)
        def _():
            fire(kB + 2, elbb, erbb, seb, srb)

    plsc.subcore_barrier()
    pltpu.sync_copy(denom_sh.at[pl.ds(s * NPR, NPR)],
                    dparts.at[c, pl.ds(s * NPR, NPR)])


def _kb_body(proj8, srcf, dst2d, w_hbm, dparts, out8,
             src_v, dst_v, gbufa, gbufb, wbufa, wbufb, table_sh, rst_sh,
             sga, sgb, swa, swb, ssa, ssb):
    c = lax.axis_index("c")
    s = lax.axis_index("s")
    kpt = EP // CH // NS

    pltpu.sync_copy(srcf.at[pl.ds(s * (kpt * CH), kpt * CH)], src_v)
    pltpu.sync_copy(dst2d.at[pl.ds(s * kpt, kpt)], dst_v)

    def fire(k, gb, wb, sg, sw):
        pltpu.async_copy(table_sh.at[src_v.at[pl.ds(k * CH, CH)]], gb, sg)
        pltpu.async_copy(w_hbm.at[pl.ds((s * kpt + k) * CH, CH)], wb, sw)

    def wait_in(k, gb, wb, sg, sw):
        pltpu.make_async_copy(table_sh.at[src_v.at[pl.ds(k * CH, CH)]],
                              gb, sg).wait()
        pltpu.make_async_copy(w_hbm.at[pl.ds((s * kpt + k) * CH, CH)],
                              wb, sw).wait()

    for hh in range(NH // NC):
        h = c * (NH // NC) + hh
        lane_h = jnp.full((16,), h, jnp.int32)

        pltpu.sync_copy(proj8.at[h, pl.ds(s * 625, 625)],
                        table_sh.at[pl.ds(s * 625, 625)])

        @pl.loop(0, CH)
        def _(j):
            for kk in range(4):
                gbufa[j, pl.ds(16 * kk, 16)] = jnp.zeros((16,), jnp.float32)

        off = 0
        for sz in (128, 128, 128, 128, 120):
            pltpu.sync_copy(gbufa.at[pl.ds(0, sz)],
                            rst_sh.at[pl.ds(s * NPR + off, sz)])
            off += sz
        plsc.subcore_barrier()

        def scale(gb, wb):
            @plsc.parallel_loop(0, CH, unroll=8)
            def _(j):
                wrow = wb[j]
                b = wrow.at[lane_h].get(mode="promise_in_bounds")
                for kk in range(4):
                    gb[j, pl.ds(16 * kk, 16)] *= b

        fire(0, gbufa, wbufa, sga, swa)
        fire(1, gbufb, wbufb, sgb, swb)

        @pl.loop(0, kpt // 2)
        def _(i):
            kA = 2 * i
            kB = kA + 1
            wait_in(kA, gbufa, wbufa, sga, swa)
            scale(gbufa, wbufa)
            pltpu.async_copy(gbufa, rst_sh.at[dst_v.at[kA]], ssa, add=True)
            wait_in(kB, gbufb, wbufb, sgb, swb)
            scale(gbufb, wbufb)
            pltpu.async_copy(gbufb, rst_sh.at[dst_v.at[kB]], ssb, add=True)
            pltpu.make_async_copy(gbufa, rst_sh.at[dst_v.at[kA]], ssa).wait()

            @pl.when(kA + 2 < kpt)
            def _():
                fire(kA + 2, gbufa, wbufa, sga, swa)

            pltpu.make_async_copy(gbufb, rst_sh.at[dst_v.at[kB]], ssb).wait()

            @pl.when(kB + 2 < kpt)
            def _():
                fire(kB + 2, gbufb, wbufb, sgb, swb)

        plsc.subcore_barrier()

        base = 0
        for sz in (80, 80, 80, 80, 80, 80, 80, 72):
            r0 = s * NPR + base
            pltpu.sync_copy(rst_sh.at[pl.ds(r0, sz)], gbufa.at[pl.ds(0, sz)])
            pltpu.sync_copy(dparts.at[0, pl.ds(r0, sz)], wbufa.at[pl.ds(0, sz)])
            pltpu.sync_copy(dparts.at[1, pl.ds(r0, sz)], wbufb.at[pl.ds(0, sz)])

            @plsc.parallel_loop(0, sz, unroll=4)
            def _(j):
                drow = wbufa[j] + wbufb[j]
                d = jnp.maximum(
                    drow.at[lane_h].get(mode="promise_in_bounds"), 1e-30)
                for kk in range(4):
                    v = gbufa[j, pl.ds(16 * kk, 16)]
                    gbufa[j, pl.ds(16 * kk, 16)] = v / d

            pltpu.sync_copy(gbufa.at[pl.ds(0, sz)], out8.at[h, pl.ds(r0, sz)])
            base += sz

        plsc.subcore_barrier()


def _sc_mesh():
    return plsc.VectorSubcoreMesh(core_axis_name="c", subcore_axis_name="s",
                                  num_cores=NC, num_subcores=NS)


_SC_PARAMS = pltpu.CompilerParams(use_tc_tiling_on_sc=False)


def _run_ka(elp_p, erp_p, srcf, dst2d):
    sds = jax.ShapeDtypeStruct
    f = pl.kernel(
        _ka_body,
        out_type=(sds((EP, 16), jnp.float32), sds((NC, NP_, 16), jnp.float32)),
        mesh=_sc_mesh(),
        scratch_types=[
            pltpu.VMEM((CPT * CH,), jnp.int32),
            pltpu.VMEM((CPT, CH), jnp.int32),
            pltpu.VMEM((CH, 16), jnp.float32),
            pltpu.VMEM((CH, 16), jnp.float32),
            pltpu.VMEM((CH, 16), jnp.float32),
            pltpu.VMEM((CH, 16), jnp.float32),
            pltpu.VMEM((NPR, 16), jnp.float32),
            pltpu.VMEM_SHARED((NP_, 16), jnp.float32),
        ] + [pltpu.SemaphoreType.DMA] * 8,
        compiler_params=_SC_PARAMS,
    )
    return f(elp_p, erp_p, srcf, dst2d)


def _run_kb(proj8, srcf, dst2d, w_e, dparts):
    sds = jax.ShapeDtypeStruct
    kpt = EP // CH // NS
    f = pl.kernel(
        _kb_body,
        out_type=sds((NH, NP_, HD), jnp.float32),
        mesh=_sc_mesh(),
        scratch_types=[
            pltpu.VMEM((kpt * CH,), jnp.int32),
            pltpu.VMEM((kpt, CH), jnp.int32),
            pltpu.VMEM((CH, HD), jnp.float32),
            pltpu.VMEM((CH, HD), jnp.float32),
            pltpu.VMEM((CH, 16), jnp.float32),
            pltpu.VMEM((CH, 16), jnp.float32),
            pltpu.VMEM_SHARED((N_NODES, HD), jnp.float32),
            pltpu.VMEM_SHARED((NP_, HD), jnp.float32),
        ] + [pltpu.SemaphoreType.DMA] * 6,
        compiler_params=_SC_PARAMS,
    )
    return f(proj8, srcf, dst2d, w_e, dparts)


def kernel(feat, edge_index, W_fc, attn_l, attn_r):
    Wt = W_fc.T
    W8 = Wt.reshape(IN_F, NH, HD).transpose(1, 0, 2)
    eye = jnp.eye(NH, dtype=jnp.float32)
    al = attn_l.reshape(NH, HD)
    ar = attn_r.reshape(NH, HD)
    ALP = jnp.pad((al[:, :, None] * eye[:, None, :]).reshape(F, NH),
                  ((0, 0), (0, 8)))
    ARP = jnp.pad((ar[:, :, None] * eye[:, None, :]).reshape(F, NH),
                  ((0, 0), (0, 8)))
    ALPARP = jnp.concatenate([ALP, ARP], axis=1)

    wlr = _tc_wl(Wt, ALPARP)
    elp_p, erp_p = _tc_eler(feat, wlr)
    proj8 = _tc_project(feat, W8)

    srcf = jnp.pad(edge_index[0], (0, EP - N_EDGES))
    dstf = jnp.pad(edge_index[1], (0, EP - N_EDGES), constant_values=N_NODES)
    dst2d = dstf.reshape(EP // CH, CH)

    w_e, dparts = _run_ka(elp_p, erp_p, srcf, dst2d)
    out8 = _run_kb(proj8, srcf, dst2d, w_e, dparts)
    return out8[:, :N_NODES].transpose(1, 0, 2)
